# baseline (speedup 1.0000x reference)
import functools

import jax
import jax.numpy as jnp
from jax import lax
from jax.experimental import pallas as pl
from jax.experimental.pallas import tpu as pltpu
from jax.experimental.pallas import tpu_sc as plsc

_NW = 32
_BPW = 512
_ROW_Y, _ROW_1 = 7, 8


def _sc_gather_body(tm, td, tw, ts, ti, mi, di, wi, si, ii,
                    em, ed, ew, es, ei,
                    idx_v, r16a, r16b, r16c, r16d, r32, sem_g, sem_w):
    wid = lax.axis_index("s") * 2 + lax.axis_index("c")
    base = wid * _BPW
    work = ((mi, tm, r16a, em), (di, td, r16b, ed), (wi, tw, r16c, ew),
            (si, ts, r16d, es), (ii, ti, r32, ei))
    outs = []
    for idx_hbm, tbl_hbm, rows_v, e_hbm in work:
        pltpu.sync_copy(idx_hbm.at[pl.ds(base, _BPW)], idx_v)
        pltpu.async_copy(tbl_hbm.at[idx_v], rows_v, sem_g).wait()
        outs.append(pltpu.async_copy(rows_v, e_hbm.at[pl.ds(base, _BPW)],
                                     sem_w))
    for h in outs:
        h.wait()


def _dgT(a, b):
    return lax.dot_general(a, b, (((0,), (0,)), ((), ())),
                           preferred_element_type=jnp.float32)


def _mlp_body(em_ref, ed_ref, ew_ref, es_ref, ei_ref, year_ref,
              w1_ref, b1_ref, w2_ref, b2_ref, w3_ref, b3_ref,
              out_ref, w1a_ref, w2e_ref, w3e_ref):
    i = pl.program_id(0)

    @pl.when(i == 0)
    def _assemble():
        w1a_ref[...] = jnp.zeros_like(w1a_ref)
        w1a_ref[0:7, 0:100] = w1_ref[1:8, :]
        w1a_ref[_ROW_Y:_ROW_Y + 1, 0:100] = w1_ref[0:1, :]
        w1a_ref[_ROW_1:_ROW_1 + 1, 0:100] = b1_ref[...][None, :]
        w1a_ref[_ROW_1:_ROW_1 + 1, 100:101] = jnp.ones((1, 1), jnp.float32)
        w1a_ref[16:32, 0:100] = w1_ref[8:24, :]
        w1a_ref[32:36, 0:100] = w1_ref[24:28, :]
        w1a_ref[48:54, 0:100] = w1_ref[28:34, :]
        w1a_ref[64:90, 0:100] = w1_ref[34:60, :]
        w2e_ref[...] = jnp.zeros_like(w2e_ref)
        w2e_ref[0:100, 0:10] = w2_ref[...]
        w2e_ref[100:101, 0:10] = b2_ref[...][None, :]
        w2e_ref[100:101, 10:11] = jnp.ones((1, 1), jnp.float32)
        w3e_ref[...] = jnp.zeros_like(w3e_ref)
        w3e_ref[0:10, 0:1] = w3_ref[...]
        w3e_ref[10:11, 0:1] = b3_ref[...][None, :]

    bblk = year_ref.shape[1]
    riota = lax.broadcasted_iota(jnp.int32, (16, bblk), 0)
    emt = em_ref[...].T
    emt = jnp.where(riota == _ROW_Y, year_ref[...], emt)
    emt = jnp.where(riota == _ROW_1, 1.0, emt)
    et = jnp.concatenate(
        [emt, ed_ref[...].T, ew_ref[...].T, es_ref[...].T, ei_ref[...].T],
        axis=0)
    h1 = jnp.maximum(_dgT(w1a_ref[...], et), 0.0)
    h2 = jnp.maximum(_dgT(w2e_ref[...], h1), 0.0)
    out_ref[...] = _dgT(w3e_ref[...], h2)


def kernel(year, month, day, weekday, stores, items, emb_month, emb_day,
           emb_weekday, emb_stores, emb_items, W1, b1, W2, b2, W3, b3):
    B = year.shape[0]

    pad = lambda t, w: jnp.pad(t, ((0, 0), (0, w - t.shape[1])))
    tm, td = pad(emb_month, 16), pad(emb_day, 16)
    tw, ts = pad(emb_weekday, 16), pad(emb_stores, 16)
    ti = pad(emb_items, 32)

    mesh = plsc.VectorSubcoreMesh(core_axis_name="c", subcore_axis_name="s")
    e16 = jax.ShapeDtypeStruct((B, 16), jnp.float32)
    sc_gather = functools.partial(
        pl.kernel, mesh=mesh,
        out_type=(e16, e16, e16, e16,
                  jax.ShapeDtypeStruct((B, 32), jnp.float32)),
        scratch_types=[pltpu.VMEM((_BPW,), jnp.int32)]
        + [pltpu.VMEM((_BPW, 16), jnp.float32)] * 4
        + [pltpu.VMEM((_BPW, 32), jnp.float32),
           pltpu.SemaphoreType.DMA, pltpu.SemaphoreType.DMA],
        compiler_params=pltpu.CompilerParams(use_tc_tiling_on_sc=False),
    )(_sc_gather_body)
    em, ed, ew, es, ei = sc_gather(
        tm, td, tw, ts, ti, month.reshape(B), day.reshape(B),
        weekday.reshape(B), stores.reshape(B), items.reshape(B))

    bblk = B
    row = pl.BlockSpec((1, bblk), lambda i: (0, i))
    eblk = lambda s: pl.BlockSpec((bblk, s[1]), lambda i: (i, 0))
    full = lambda s: pl.BlockSpec(s, lambda i: (0,) * len(s))
    out = pl.pallas_call(
        _mlp_body,
        grid=(B // bblk,),
        in_specs=[eblk((B, 16))] * 4 + [eblk((B, 32)), row,
                  full(W1.shape), full(b1.shape), full(W2.shape),
                  full(b2.shape), full(W3.shape), full(b3.shape)],
        out_specs=row,
        out_shape=jax.ShapeDtypeStruct((1, B), jnp.float32),
        scratch_shapes=[pltpu.VMEM((96, 104), jnp.float32),
                        pltpu.VMEM((104, 16), jnp.float32),
                        pltpu.VMEM((16, 1), jnp.float32)],
        compiler_params=pltpu.CompilerParams(
            dimension_semantics=("arbitrary",)),
    )(em, ed, ew, es, ei, year.reshape(1, B),
      W1, b1, W2, b2, W3, b3)
    return out.reshape(B, 1)


# speedup vs baseline: 2.7788x; 2.7788x over previous
import functools

import jax
import jax.numpy as jnp
from jax import lax
from jax.experimental import pallas as pl
from jax.experimental.pallas import tpu as pltpu
from jax.experimental.pallas import tpu_sc as plsc

_NW = 32
_BPW = 512
_ROW_Y, _ROW_1 = 7, 8


def _sc_gather_body(tm, td, tw, ts, ti, mi, di, wi, si, ii,
                    em, ed, ew, es, ei,
                    idx_v, r16a, r16b, r16c, r16d, r32,
                    tmv, tdv, twv, tsv, tiv, sem_g, sem_w):
    sid = lax.axis_index("s")
    wid = sid * 2 + lax.axis_index("c")
    base = wid * _BPW

    @pl.when(sid == 0)
    def _stage():
        for tbl_hbm, tbl_v in ((tm, tmv), (td, tdv), (tw, twv), (ts, tsv),
                               (ti, tiv)):
            pltpu.sync_copy(tbl_hbm, tbl_v)

    plsc.subcore_barrier()
    work = ((mi, tmv, r16a, em), (di, tdv, r16b, ed), (wi, twv, r16c, ew),
            (si, tsv, r16d, es), (ii, tiv, r32, ei))
    outs = []
    for idx_hbm, tbl_v, rows_v, e_hbm in work:
        pltpu.sync_copy(idx_hbm.at[pl.ds(base, _BPW)], idx_v)
        pltpu.async_copy(tbl_v.at[idx_v], rows_v, sem_g).wait()
        outs.append(pltpu.async_copy(rows_v, e_hbm.at[pl.ds(base, _BPW)],
                                     sem_w))
    for h in outs:
        h.wait()


def _dgT(a, b):
    return lax.dot_general(a, b, (((0,), (0,)), ((), ())),
                           preferred_element_type=jnp.float32)


def _mlp_body(em_ref, ed_ref, ew_ref, es_ref, ei_ref, year_ref,
              w1_ref, b1_ref, w2_ref, b2_ref, w3_ref, b3_ref,
              out_ref, w1a_ref, w2e_ref, w3e_ref):
    i = pl.program_id(0)

    @pl.when(i == 0)
    def _assemble():
        w1a_ref[...] = jnp.zeros_like(w1a_ref)
        w1a_ref[0:7, 0:100] = w1_ref[1:8, :]
        w1a_ref[_ROW_Y:_ROW_Y + 1, 0:100] = w1_ref[0:1, :]
        w1a_ref[_ROW_1:_ROW_1 + 1, 0:100] = b1_ref[...][None, :]
        w1a_ref[_ROW_1:_ROW_1 + 1, 100:101] = jnp.ones((1, 1), jnp.float32)
        w1a_ref[16:32, 0:100] = w1_ref[8:24, :]
        w1a_ref[32:36, 0:100] = w1_ref[24:28, :]
        w1a_ref[48:54, 0:100] = w1_ref[28:34, :]
        w1a_ref[64:90, 0:100] = w1_ref[34:60, :]
        w2e_ref[...] = jnp.zeros_like(w2e_ref)
        w2e_ref[0:100, 0:10] = w2_ref[...]
        w2e_ref[100:101, 0:10] = b2_ref[...][None, :]
        w2e_ref[100:101, 10:11] = jnp.ones((1, 1), jnp.float32)
        w3e_ref[...] = jnp.zeros_like(w3e_ref)
        w3e_ref[0:10, 0:1] = w3_ref[...]
        w3e_ref[10:11, 0:1] = b3_ref[...][None, :]

    bblk = year_ref.shape[1]
    riota = lax.broadcasted_iota(jnp.int32, (16, bblk), 0)
    emt = em_ref[...].T
    emt = jnp.where(riota == _ROW_Y, year_ref[...], emt)
    emt = jnp.where(riota == _ROW_1, 1.0, emt)
    et = jnp.concatenate(
        [emt, ed_ref[...].T, ew_ref[...].T, es_ref[...].T, ei_ref[...].T],
        axis=0)
    h1 = jnp.maximum(_dgT(w1a_ref[...], et), 0.0)
    h2 = jnp.maximum(_dgT(w2e_ref[...], h1), 0.0)
    out_ref[...] = _dgT(w3e_ref[...], h2)


def kernel(year, month, day, weekday, stores, items, emb_month, emb_day,
           emb_weekday, emb_stores, emb_items, W1, b1, W2, b2, W3, b3):
    B = year.shape[0]

    pad = lambda t, w: jnp.pad(t, ((0, 0), (0, w - t.shape[1])))
    tm, td = pad(emb_month, 16), pad(emb_day, 16)
    tw, ts = pad(emb_weekday, 16), pad(emb_stores, 16)
    ti = pad(emb_items, 32)

    mesh = plsc.VectorSubcoreMesh(core_axis_name="c", subcore_axis_name="s")
    e16 = jax.ShapeDtypeStruct((B, 16), jnp.float32)
    sc_gather = functools.partial(
        pl.kernel, mesh=mesh,
        out_type=(e16, e16, e16, e16,
                  jax.ShapeDtypeStruct((B, 32), jnp.float32)),
        scratch_types=[pltpu.VMEM((_BPW,), jnp.int32)]
        + [pltpu.VMEM((_BPW, 16), jnp.float32)] * 4
        + [pltpu.VMEM((_BPW, 32), jnp.float32),
           pltpu.VMEM_SHARED((13, 16), jnp.float32),
           pltpu.VMEM_SHARED((32, 16), jnp.float32),
           pltpu.VMEM_SHARED((8, 16), jnp.float32),
           pltpu.VMEM_SHARED((11, 16), jnp.float32),
           pltpu.VMEM_SHARED((51, 32), jnp.float32),
           pltpu.SemaphoreType.DMA, pltpu.SemaphoreType.DMA],
        compiler_params=pltpu.CompilerParams(use_tc_tiling_on_sc=False),
    )(_sc_gather_body)
    em, ed, ew, es, ei = sc_gather(
        tm, td, tw, ts, ti, month.reshape(B), day.reshape(B),
        weekday.reshape(B), stores.reshape(B), items.reshape(B))

    bblk = B
    row = pl.BlockSpec((1, bblk), lambda i: (0, i))
    eblk = lambda s: pl.BlockSpec((bblk, s[1]), lambda i: (i, 0))
    full = lambda s: pl.BlockSpec(s, lambda i: (0,) * len(s))
    out = pl.pallas_call(
        _mlp_body,
        grid=(B // bblk,),
        in_specs=[eblk((B, 16))] * 4 + [eblk((B, 32)), row,
                  full(W1.shape), full(b1.shape), full(W2.shape),
                  full(b2.shape), full(W3.shape), full(b3.shape)],
        out_specs=row,
        out_shape=jax.ShapeDtypeStruct((1, B), jnp.float32),
        scratch_shapes=[pltpu.VMEM((96, 104), jnp.float32),
                        pltpu.VMEM((104, 16), jnp.float32),
                        pltpu.VMEM((16, 1), jnp.float32)],
        compiler_params=pltpu.CompilerParams(
            dimension_semantics=("arbitrary",)),
    )(em, ed, ew, es, ei, year.reshape(1, B),
      W1, b1, W2, b2, W3, b3)
    return out.reshape(B, 1)


# speedup vs baseline: 2.9292x; 1.0541x over previous
import functools

import jax
import jax.numpy as jnp
from jax import lax
from jax.experimental import pallas as pl
from jax.experimental.pallas import tpu as pltpu
from jax.experimental.pallas import tpu_sc as plsc

_NW = 32
_BPW = 512
_ROW_Y, _ROW_1 = 7, 8


def _sc_gather_body(tm, td, tw, ts, ti, mi, di, wi, si, ii,
                    em, ed, ew, es, ei,
                    ia, ib, ic, id_, ie, r16a, r16b, r16c, r16d, r32,
                    tmv, tdv, twv, tsv, tiv, sem_i, sem_g, sem_w):
    sid = lax.axis_index("s")
    wid = sid * 2 + lax.axis_index("c")
    base = wid * _BPW

    @pl.when(sid == 0)
    def _stage():
        hs = [pltpu.async_copy(tbl_hbm, tbl_v, sem_i)
              for tbl_hbm, tbl_v in ((tm, tmv), (td, tdv), (tw, twv),
                                     (ts, tsv), (ti, tiv))]
        for h in hs:
            h.wait()

    plsc.subcore_barrier()
    work = ((mi, ia, tmv, r16a, em), (di, ib, tdv, r16b, ed),
            (wi, ic, twv, r16c, ew), (si, id_, tsv, r16d, es),
            (ii, ie, tiv, r32, ei))
    hs = [pltpu.async_copy(idx_hbm.at[pl.ds(base, _BPW)], idx_v, sem_i)
          for idx_hbm, idx_v, _, _, _ in work]
    for h in hs:
        h.wait()
    hs = [pltpu.async_copy(tbl_v.at[idx_v], rows_v, sem_g)
          for _, idx_v, tbl_v, rows_v, _ in work]
    for h in hs:
        h.wait()
    hs = [pltpu.async_copy(rows_v, e_hbm.at[pl.ds(base, _BPW)], sem_w)
          for _, _, _, rows_v, e_hbm in work]
    for h in hs:
        h.wait()


def _dgT(a, b):
    return lax.dot_general(a, b, (((0,), (0,)), ((), ())),
                           preferred_element_type=jnp.float32)


def _mlp_body(em_ref, ed_ref, ew_ref, es_ref, ei_ref, year_ref,
              w1_ref, b1_ref, w2_ref, b2_ref, w3_ref, b3_ref,
              out_ref, w1a_ref, w2e_ref, w3e_ref):
    i = pl.program_id(0)

    @pl.when(i == 0)
    def _assemble():
        w1a_ref[...] = jnp.zeros_like(w1a_ref)
        w1a_ref[0:7, 0:100] = w1_ref[1:8, :]
        w1a_ref[_ROW_Y:_ROW_Y + 1, 0:100] = w1_ref[0:1, :]
        w1a_ref[_ROW_1:_ROW_1 + 1, 0:100] = b1_ref[...][None, :]
        w1a_ref[_ROW_1:_ROW_1 + 1, 100:101] = jnp.ones((1, 1), jnp.float32)
        w1a_ref[16:32, 0:100] = w1_ref[8:24, :]
        w1a_ref[32:36, 0:100] = w1_ref[24:28, :]
        w1a_ref[48:54, 0:100] = w1_ref[28:34, :]
        w1a_ref[64:90, 0:100] = w1_ref[34:60, :]
        w2e_ref[...] = jnp.zeros_like(w2e_ref)
        w2e_ref[0:100, 0:10] = w2_ref[...]
        w2e_ref[100:101, 0:10] = b2_ref[...][None, :]
        w2e_ref[100:101, 10:11] = jnp.ones((1, 1), jnp.float32)
        w3e_ref[...] = jnp.zeros_like(w3e_ref)
        w3e_ref[0:10, 0:1] = w3_ref[...]
        w3e_ref[10:11, 0:1] = b3_ref[...][None, :]

    bblk = year_ref.shape[1]
    riota = lax.broadcasted_iota(jnp.int32, (16, bblk), 0)
    emt = em_ref[...].T
    emt = jnp.where(riota == _ROW_Y, year_ref[...], emt)
    emt = jnp.where(riota == _ROW_1, 1.0, emt)
    et = jnp.concatenate(
        [emt, ed_ref[...].T, ew_ref[...].T, es_ref[...].T, ei_ref[...].T],
        axis=0)
    h1 = jnp.maximum(_dgT(w1a_ref[...], et), 0.0)
    h2 = jnp.maximum(_dgT(w2e_ref[...], h1), 0.0)
    out_ref[...] = _dgT(w3e_ref[...], h2)


def kernel(year, month, day, weekday, stores, items, emb_month, emb_day,
           emb_weekday, emb_stores, emb_items, W1, b1, W2, b2, W3, b3):
    B = year.shape[0]

    pad = lambda t, w: jnp.pad(t, ((0, 0), (0, w - t.shape[1])))
    tm, td = pad(emb_month, 16), pad(emb_day, 16)
    tw, ts = pad(emb_weekday, 16), pad(emb_stores, 16)
    ti = pad(emb_items, 32)

    mesh = plsc.VectorSubcoreMesh(core_axis_name="c", subcore_axis_name="s")
    e16 = jax.ShapeDtypeStruct((B, 16), jnp.float32)
    sc_gather = functools.partial(
        pl.kernel, mesh=mesh,
        out_type=(e16, e16, e16, e16,
                  jax.ShapeDtypeStruct((B, 32), jnp.float32)),
        scratch_types=[pltpu.VMEM((_BPW,), jnp.int32)] * 5
        + [pltpu.VMEM((_BPW, 16), jnp.float32)] * 4
        + [pltpu.VMEM((_BPW, 32), jnp.float32),
           pltpu.VMEM_SHARED((13, 16), jnp.float32),
           pltpu.VMEM_SHARED((32, 16), jnp.float32),
           pltpu.VMEM_SHARED((8, 16), jnp.float32),
           pltpu.VMEM_SHARED((11, 16), jnp.float32),
           pltpu.VMEM_SHARED((51, 32), jnp.float32),
           pltpu.SemaphoreType.DMA, pltpu.SemaphoreType.DMA,
           pltpu.SemaphoreType.DMA],
        compiler_params=pltpu.CompilerParams(use_tc_tiling_on_sc=False),
    )(_sc_gather_body)
    em, ed, ew, es, ei = sc_gather(
        tm, td, tw, ts, ti, month.reshape(B), day.reshape(B),
        weekday.reshape(B), stores.reshape(B), items.reshape(B))

    bblk = B
    row = pl.BlockSpec((1, bblk), lambda i: (0, i))
    eblk = lambda s: pl.BlockSpec((bblk, s[1]), lambda i: (i, 0))
    full = lambda s: pl.BlockSpec(s, lambda i: (0,) * len(s))
    out = pl.pallas_call(
        _mlp_body,
        grid=(B // bblk,),
        in_specs=[eblk((B, 16))] * 4 + [eblk((B, 32)), row,
                  full(W1.shape), full(b1.shape), full(W2.shape),
                  full(b2.shape), full(W3.shape), full(b3.shape)],
        out_specs=row,
        out_shape=jax.ShapeDtypeStruct((1, B), jnp.float32),
        scratch_shapes=[pltpu.VMEM((96, 104), jnp.float32),
                        pltpu.VMEM((104, 16), jnp.float32),
                        pltpu.VMEM((16, 1), jnp.float32)],
        compiler_params=pltpu.CompilerParams(
            dimension_semantics=("arbitrary",)),
    )(em, ed, ew, es, ei, year.reshape(1, B),
      W1, b1, W2, b2, W3, b3)
    return out.reshape(B, 1)
